# baseline (speedup 1.0000x reference)
import jax
import jax.numpy as jnp
from jax.experimental import pallas as pl
from jax.experimental.pallas import tpu as pltpu

_NT = 4
_TM = 512


def _make_mlp_kernel(M, I, H, O):
    nb = pl.cdiv(M, _TM)
    blocks = [(b * _TM, min(_TM, M - b * _TM)) for b in range(nb)]
    r1 = I // _NT
    r2 = H // _NT

    def _mlp_kernel(x_hbm, b1_ref, w1_hbm, w2_hbm, b2_ref, o_hbm,
                    w1s, w2s, xstg, ostg, sem1, sem2, semx, semo):
        def cw1(t):
            return pltpu.make_async_copy(
                w1_hbm.at[pl.ds(t * r1, r1), :], w1s.at[pl.ds(t * r1, r1), :],
                sem1.at[t])

        def cw2(t):
            return pltpu.make_async_copy(
                w2_hbm.at[pl.ds(t * r2, r2), :], w2s.at[pl.ds(t * r2, r2), :],
                sem2.at[t])

        def cx(b):
            base, rows = blocks[b]
            return pltpu.make_async_copy(
                x_hbm.at[pl.ds(base, rows), :],
                xstg.at[b % 3, pl.ds(0, rows), :], semx.at[b % 3])

        def co(b):
            base, rows = blocks[b]
            return pltpu.make_async_copy(
                ostg.at[b % 2, pl.ds(0, rows), :],
                o_hbm.at[pl.ds(base, rows), :], semo.at[b % 2])

        for t in range(_NT):
            cw1(t).start()
        for t in range(_NT):
            cw2(t).start()
        cx(0).start()
        if nb > 1:
            cx(1).start()

        for b in range(nb):
            base, rows = blocks[b]
            if b + 2 < nb:
                cx(b + 2).start()
            cx(b).wait()
            x = xstg[b % 3, pl.ds(0, rows), :]
            if b == 0:
                h = b1_ref[...] * jnp.ones((rows, 1), jnp.float32)
                for t in range(_NT):
                    cw1(t).wait()
                    h = h + jnp.dot(x[:, t * r1:(t + 1) * r1],
                                    w1s[pl.ds(t * r1, r1), :],
                                    preferred_element_type=jnp.float32)
                h = jnp.maximum(h, 0.0)
                out = b2_ref[...] * jnp.ones((rows, 1), jnp.float32)
                for t in range(_NT):
                    cw2(t).wait()
                    out = out + jnp.dot(h[:, t * r2:(t + 1) * r2],
                                        w2s[pl.ds(t * r2, r2), :],
                                        preferred_element_type=jnp.float32)
            else:
                h = jnp.dot(x, w1s[...], preferred_element_type=jnp.float32)
                h = jnp.maximum(h + b1_ref[...], 0.0)
                out = jnp.dot(h, w2s[...],
                              preferred_element_type=jnp.float32) + b2_ref[...]
            if b >= 2:
                co(b - 2).wait()
            ostg[b % 2, pl.ds(0, rows), :] = out
            co(b).start()

        if nb >= 2:
            co(nb - 2).wait()
        co(nb - 1).wait()

    return _mlp_kernel


def kernel(x, w1, b1, w2, b2):
    I = x.shape[-1]
    H = w1.shape[1]
    O = w2.shape[1]
    lead_shape = x.shape[:-1]

    x2 = x.reshape(-1, I)
    M = x2.shape[0]
    tm = min(_TM, M)

    b1r = b1.reshape(1, H)
    b2r = b2.reshape(1, O)

    working = (4 * (I * H + H * O)
               + 4 * tm * (3 * I + 2 * O)
               + 4 * tm * H
               + 4 * (H + O))
    vmem_limit = int(min(max(working + 8 * 1024 * 1024, 4 * 1024 * 1024),
                         58 * 1024 * 1024))

    cost = pl.CostEstimate(
        flops=2 * M * (I * H + H * O),
        transcendentals=0,
        bytes_accessed=4 * (M * I + I * H + H + H * O + O + M * O),
    )

    out = pl.pallas_call(
        _make_mlp_kernel(M, I, H, O),
        out_shape=jax.ShapeDtypeStruct((M, O), x.dtype),
        grid=(1,),
        in_specs=[
            pl.BlockSpec(memory_space=pl.ANY),
            pl.BlockSpec((1, H), lambda i: (0, 0)),
            pl.BlockSpec(memory_space=pl.ANY),
            pl.BlockSpec(memory_space=pl.ANY),
            pl.BlockSpec((1, O), lambda i: (0, 0)),
        ],
        out_specs=pl.BlockSpec(memory_space=pl.ANY),
        scratch_shapes=[
            pltpu.VMEM((I, H), jnp.float32),
            pltpu.VMEM((H, O), jnp.float32),
            pltpu.VMEM((3, tm, I), jnp.float32),
            pltpu.VMEM((2, tm, O), jnp.float32),
            pltpu.SemaphoreType.DMA((_NT,)),
            pltpu.SemaphoreType.DMA((_NT,)),
            pltpu.SemaphoreType.DMA((3,)),
            pltpu.SemaphoreType.DMA((2,)),
        ],
        compiler_params=pltpu.CompilerParams(
            dimension_semantics=("arbitrary",),
            vmem_limit_bytes=vmem_limit,
        ),
        cost_estimate=cost,
    )(x2, b1r, w1, w2, b2r)

    return out.reshape(*lead_shape, O)


# speedup vs baseline: 1.0454x; 1.0454x over previous
import jax
import jax.numpy as jnp
from jax.experimental import pallas as pl
from jax.experimental.pallas import tpu as pltpu

_NT = 4


def _mlp_kernel(x_ref, w1_hbm, b1_ref, w2_hbm, b2_ref, o_ref,
                w1s, w2s, sem1, sem2):
    I, H = w1s.shape
    O = w2s.shape[1]
    r1 = I // _NT
    r2 = H // _NT
    i = pl.program_id(0)

    def c1(t):
        return pltpu.make_async_copy(
            w1_hbm.at[pl.ds(t * r1, r1), :], w1s.at[pl.ds(t * r1, r1), :],
            sem1.at[t])

    def c2(t):
        return pltpu.make_async_copy(
            w2_hbm.at[pl.ds(t * r2, r2), :], w2s.at[pl.ds(t * r2, r2), :],
            sem2.at[t])

    @pl.when(i == 0)
    def _first_step():
        for t in range(_NT):
            c1(t).start()
        for t in range(_NT):
            c2(t).start()
        x = x_ref[...]
        h = b1_ref[...] * jnp.ones((x.shape[0], 1), jnp.float32)
        for t in range(_NT):
            c1(t).wait()
            h = h + jnp.dot(x[:, t * r1:(t + 1) * r1],
                            w1s[pl.ds(t * r1, r1), :],
                            preferred_element_type=jnp.float32)
        h = jnp.maximum(h, 0.0)
        acc = b2_ref[...] * jnp.ones((x.shape[0], 1), jnp.float32)
        for t in range(_NT):
            c2(t).wait()
            acc = acc + jnp.dot(h[:, t * r2:(t + 1) * r2],
                                w2s[pl.ds(t * r2, r2), :],
                                preferred_element_type=jnp.float32)
        o_ref[...] = acc.astype(o_ref.dtype)

    @pl.when(i > 0)
    def _steady_state():
        h = jnp.dot(x_ref[...], w1s[...], preferred_element_type=jnp.float32)
        h = jnp.maximum(h + b1_ref[...], 0.0)
        out = jnp.dot(h, w2s[...],
                      preferred_element_type=jnp.float32) + b2_ref[...]
        o_ref[...] = out.astype(o_ref.dtype)


def kernel(x, w1, b1, w2, b2):
    I = x.shape[-1]
    H = w1.shape[1]
    O = w2.shape[1]
    lead_shape = x.shape[:-1]

    x2 = x.reshape(-1, I)
    M = x2.shape[0]

    if M <= 512:
        tm = M
    else:
        tm = 512
    grid_m = pl.cdiv(M, tm)

    b1r = b1.reshape(1, H)
    b2r = b2.reshape(1, O)

    working = (4 * (I * H + H * O)
               + 2 * 4 * (tm * I + tm * O)
               + 4 * (tm * H + H + O))
    vmem_limit = int(min(max(working + 8 * 1024 * 1024, 4 * 1024 * 1024),
                         58 * 1024 * 1024))

    cost = pl.CostEstimate(
        flops=2 * M * (I * H + H * O),
        transcendentals=0,
        bytes_accessed=4 * (M * I + I * H + H + H * O + O + M * O),
    )

    out = pl.pallas_call(
        _mlp_kernel,
        out_shape=jax.ShapeDtypeStruct((M, O), x.dtype),
        grid=(grid_m,),
        in_specs=[
            pl.BlockSpec((tm, I), lambda i: (i, 0)),
            pl.BlockSpec(memory_space=pl.ANY),
            pl.BlockSpec((1, H), lambda i: (0, 0)),
            pl.BlockSpec(memory_space=pl.ANY),
            pl.BlockSpec((1, O), lambda i: (0, 0)),
        ],
        out_specs=pl.BlockSpec((tm, O), lambda i: (i, 0)),
        scratch_shapes=[
            pltpu.VMEM((I, H), jnp.float32),
            pltpu.VMEM((H, O), jnp.float32),
            pltpu.SemaphoreType.DMA((_NT,)),
            pltpu.SemaphoreType.DMA((_NT,)),
        ],
        compiler_params=pltpu.CompilerParams(
            dimension_semantics=("arbitrary",),
            vmem_limit_bytes=vmem_limit,
        ),
        cost_estimate=cost,
    )(x2, w1, b1r, w2, b2r)

    return out.reshape(*lead_shape, O)
